# baseline (speedup 1.0000x reference)
import functools

import jax
import jax.numpy as jnp
from jax import lax
from jax.experimental import pallas as pl
from jax.experimental.pallas import tpu as pltpu
from jax.experimental.pallas import tpu_sc as plsc

_N = 10000
_D = 128
_C = 64
_K = 8
_ALPHA = 0.05

_NC = 2
_NS = 16
_NW = _NC * _NS
_NPAD = 10240
_RPT = _NPAD // _NS
_ZR = 32

_CB = 128
_NCH = 81
_EPAD = _NW * _NCH * _CB

_CK = [(1.0 - _ALPHA) * float(_K) ** (k + 1 - _K - 1) for k in range(_K)]
_CF = _ALPHA * sum(float(_K) ** (k + 1 - _K - 1) for k in range(_K))

_mesh = plsc.VectorSubcoreMesh(core_axis_name="c", subcore_axis_name="s")


def _deg_body(dst_hbm, out_hbm, degsh, dst_v, ones_v, zb):
    c = lax.axis_index("c")
    s = lax.axis_index("s")
    w = c * _NS + s
    ones16 = jnp.ones((16,), jnp.float32)
    zeros16 = jnp.zeros((16,), jnp.float32)
    for i in range(_CB // 16):
        ones_v[pl.ds(i * 16, 16)] = ones16
    for i in range(_RPT // 16):
        zb[pl.ds(i * 16, 16)] = zeros16
    pltpu.sync_copy(zb, degsh.at[pl.ds(s * _RPT, _RPT)])
    pltpu.sync_copy(dst_hbm.at[w], dst_v)
    plsc.subcore_barrier()
    for j in range(_NCH):
        pltpu.sync_copy(ones_v, degsh.at[dst_v.at[j]], add=True)
    plsc.subcore_barrier()
    pltpu.sync_copy(degsh.at[pl.ds(s * _RPT, _RPT)],
                    out_hbm.at[c, pl.ds(s * _RPT, _RPT)])


_deg_call = pl.kernel(
    _deg_body,
    out_type=jax.ShapeDtypeStruct((_NC, _NPAD), jnp.float32),
    mesh=_mesh,
    scratch_types=[
        pltpu.VMEM_SHARED((_NPAD,), jnp.float32),
        pltpu.VMEM((_NCH, _CB), jnp.int32),
        pltpu.VMEM((_CB,), jnp.float32),
        pltpu.VMEM((_RPT,), jnp.float32),
    ],
    compiler_params=pltpu.CompilerParams(use_tc_tiling_on_sc=False),
)


def _edge_body(z_hbm, src_hbm, dst_hbm, out_hbm,
               ssh, src_v, dst_v, g0, g1, g2, g3, g4, g5, zb,
               semg, sems):
    c = lax.axis_index("c")
    s = lax.axis_index("s")
    w = c * _NS + s
    zeros16 = jnp.zeros((16,), jnp.float32)
    for r in range(_ZR):
        for q in range(_C // 16):
            zb[r, pl.ds(q * 16, 16)] = zeros16
    for blk in range(_RPT // _ZR):
        pltpu.sync_copy(zb, ssh.at[pl.ds(s * _RPT + blk * _ZR, _ZR)])
    pltpu.sync_copy(src_hbm.at[w], src_v)
    pltpu.sync_copy(dst_hbm.at[w], dst_v)
    plsc.subcore_barrier()
    gbufs = (g0, g1, g2, g3, g4, g5)
    nb = len(gbufs)
    gq = 3

    def gfire(j):
        return pltpu.async_copy(z_hbm.at[src_v.at[j]], gbufs[j % nb], semg)

    def sfire(j):
        return pltpu.async_copy(gbufs[j % nb], ssh.at[dst_v.at[j]], sems,
                                add=True)

    gd = {j: gfire(j) for j in range(min(gq, _NCH))}
    sd = {}
    for j in range(_NCH):
        gd.pop(j).wait()
        if j + gq < _NCH:
            if j + gq - nb >= 0:
                sd.pop(j + gq - nb).wait()
            gd[j + gq] = gfire(j + gq)
        sd[j] = sfire(j)
    for j in sorted(sd):
        sd.pop(j).wait()
    plsc.subcore_barrier()
    pltpu.sync_copy(ssh.at[pl.ds(s * _RPT, _RPT)],
                    out_hbm.at[c, pl.ds(s * _RPT, _RPT)])


_edge_call = pl.kernel(
    _edge_body,
    out_type=jax.ShapeDtypeStruct((_NC, _NPAD, _C), jnp.float32),
    mesh=_mesh,
    scratch_types=[
        pltpu.VMEM_SHARED((_NPAD, _C), jnp.float32),
        pltpu.VMEM((_NCH, _CB), jnp.int32),
        pltpu.VMEM((_NCH, _CB), jnp.int32),
        pltpu.VMEM((_CB, _C), jnp.float32),
        pltpu.VMEM((_CB, _C), jnp.float32),
        pltpu.VMEM((_CB, _C), jnp.float32),
        pltpu.VMEM((_CB, _C), jnp.float32),
        pltpu.VMEM((_CB, _C), jnp.float32),
        pltpu.VMEM((_CB, _C), jnp.float32),
        pltpu.VMEM((_ZR, _C), jnp.float32),
        pltpu.SemaphoreType.DMA,
        pltpu.SemaphoreType.DMA,
    ],
    compiler_params=pltpu.CompilerParams(use_tc_tiling_on_sc=False),
)


_RPW = _NPAD // _NW
_NBK = _RPW // _ZR


def _mix_body(p_hbm, d2_hbm, dv_hbm, acc_hbm, z_out, acc_out,
              pb0, pb1, d2b, dvb, acb, zwb, awb, semp, semz, *, ck):
    c = lax.axis_index("c")
    s = lax.axis_index("s")
    w = c * _NS + s

    def pfire(blk, slot):
        r0 = w * _RPW + blk * _ZR
        return (
            pltpu.async_copy(p_hbm.at[0, pl.ds(r0, _ZR)], pb0.at[slot], semp),
            pltpu.async_copy(p_hbm.at[1, pl.ds(r0, _ZR)], pb1.at[slot], semp),
            pltpu.async_copy(d2_hbm.at[pl.ds(r0, _ZR)], d2b.at[slot], semp),
            pltpu.async_copy(dv_hbm.at[pl.ds(r0, _ZR)], dvb.at[slot], semp),
            pltpu.async_copy(acc_hbm.at[pl.ds(r0, _ZR)], acb.at[slot], semp),
        )

    pend = pfire(0, 0)
    zpend = []
    for blk in range(_NBK):
        for d in pend:
            d.wait()
        slot = blk % 2
        if blk + 1 < _NBK:
            pend = pfire(blk + 1, 1 - slot)
        if blk >= 2:
            for d in zpend[blk - 2]:
                d.wait()

        def rowbody(rh, carry, slot=slot):
            for u in range(2):
                r = rh * 2 + u
                for q in range(_C // 16):
                    sl = pl.ds(q * 16, 16)
                    t = pb0[slot, r, sl] + pb1[slot, r, sl]
                    zwb[slot, r, sl] = d2b[slot, r, sl] * t
                    awb[slot, r, sl] = acb[slot, r, sl] + (
                        dvb[slot, r, sl] * t) * ck
            return carry

        lax.fori_loop(0, _ZR // 2, rowbody, 0)
        r0 = w * _RPW + blk * _ZR
        zpend.append((
            pltpu.async_copy(zwb.at[slot], z_out.at[pl.ds(r0, _ZR)], semz),
            pltpu.async_copy(awb.at[slot], acc_out.at[pl.ds(r0, _ZR)], semz),
        ))
    for pair in zpend[-2:]:
        for d in pair:
            d.wait()


def _make_mix(ck):
    return pl.kernel(
        functools.partial(_mix_body, ck=ck),
        out_type=(
            jax.ShapeDtypeStruct((_NPAD, _C), jnp.float32),
            jax.ShapeDtypeStruct((_NPAD, _C), jnp.float32),
        ),
        mesh=_mesh,
        scratch_types=[
            pltpu.VMEM((2, _ZR, _C), jnp.float32),
            pltpu.VMEM((2, _ZR, _C), jnp.float32),
            pltpu.VMEM((2, _ZR, _C), jnp.float32),
            pltpu.VMEM((2, _ZR, _C), jnp.float32),
            pltpu.VMEM((2, _ZR, _C), jnp.float32),
            pltpu.VMEM((2, _ZR, _C), jnp.float32),
            pltpu.VMEM((2, _ZR, _C), jnp.float32),
            pltpu.SemaphoreType.DMA,
            pltpu.SemaphoreType.DMA,
        ],
        compiler_params=pltpu.CompilerParams(use_tc_tiling_on_sc=False),
    )


_mix_calls = [_make_mix(ck) for ck in _CK]


def _prep_body(feat_ref, wt_ref, degp_ref, b_ref,
               z0_ref, d2ex_ref, dvex_ref, acc0_ref):
    deg = jnp.maximum(degp_ref[0] + degp_ref[1], 1.0)
    dinv = lax.rsqrt(deg)
    y0 = jnp.dot(feat_ref[...], wt_ref[...], preferred_element_type=jnp.float32)
    z0_ref[...] = y0 * dinv
    zc = jnp.zeros((_NPAD, _C), jnp.float32)
    d2ex_ref[...] = zc + 1.0 / deg
    dvex_ref[...] = zc + dinv
    acc0_ref[...] = _CF * y0 + b_ref[...]


def _prep_call(feat_pad, wt, degp3, b2d):
    return pl.pallas_call(
        _prep_body,
        out_shape=(
            jax.ShapeDtypeStruct((_NPAD, _C), jnp.float32),
            jax.ShapeDtypeStruct((_NPAD, _C), jnp.float32),
            jax.ShapeDtypeStruct((_NPAD, _C), jnp.float32),
            jax.ShapeDtypeStruct((_NPAD, _C), jnp.float32),
        ),
    )(feat_pad, wt, degp3, b2d)


@jax.jit
def kernel(feat, edge_index, W, b):
    feat_pad = jnp.zeros((_NPAD, _D), jnp.float32).at[:_N].set(feat)
    loop = jnp.arange(_N, dtype=jnp.int32)
    npadrows = _NPAD - _N
    e_in = edge_index.shape[1]
    padi = _N + (jnp.arange(_EPAD - _N - e_in, dtype=jnp.int32) % npadrows)
    src = jnp.concatenate([edge_index[0], loop, padi]).reshape(_NW, _NCH, _CB)
    dst = jnp.concatenate([edge_index[1], loop, padi]).reshape(_NW, _NCH, _CB)

    degp = _deg_call(dst)
    z0, d2ex, dvex, acc = _prep_call(feat_pad, W.T, degp[:, :, None],
                                     jnp.reshape(b, (1, _C)))

    z = z0
    for k in range(_K):
        s_k = _edge_call(z, src, dst)
        z, acc = _mix_calls[k](s_k, d2ex, dvex, acc)

    return acc[:_N]


# speedup vs baseline: 1.0202x; 1.0202x over previous
import functools

import jax
import jax.numpy as jnp
from jax import lax
from jax.experimental import pallas as pl
from jax.experimental.pallas import tpu as pltpu
from jax.experimental.pallas import tpu_sc as plsc

_N = 10000
_D = 128
_C = 64
_K = 8
_ALPHA = 0.05

_NC = 2
_NS = 16
_NW = _NC * _NS
_NPAD = 10240
_RPT = _NPAD // _NS
_ZR = 32

_CB = 128
_NCH = 81
_EPAD = _NW * _NCH * _CB

_CK = [(1.0 - _ALPHA) * float(_K) ** (k + 1 - _K - 1) for k in range(_K)]
_CF = _ALPHA * sum(float(_K) ** (k + 1 - _K - 1) for k in range(_K))

_mesh = plsc.VectorSubcoreMesh(core_axis_name="c", subcore_axis_name="s")


def _deg_body(dst_hbm, out_hbm, degsh, dst_v, ones_v, zb):
    c = lax.axis_index("c")
    s = lax.axis_index("s")
    w = c * _NS + s
    ones16 = jnp.ones((16,), jnp.float32)
    zeros16 = jnp.zeros((16,), jnp.float32)
    for i in range(_CB // 16):
        ones_v[pl.ds(i * 16, 16)] = ones16
    for i in range(_RPT // 16):
        zb[pl.ds(i * 16, 16)] = zeros16
    pltpu.sync_copy(zb, degsh.at[pl.ds(s * _RPT, _RPT)])
    pltpu.sync_copy(dst_hbm.at[w], dst_v)
    plsc.subcore_barrier()
    for j in range(_NCH):
        pltpu.sync_copy(ones_v, degsh.at[dst_v.at[j]], add=True)
    plsc.subcore_barrier()
    pltpu.sync_copy(degsh.at[pl.ds(s * _RPT, _RPT)],
                    out_hbm.at[c, pl.ds(s * _RPT, _RPT)])


_deg_call = pl.kernel(
    _deg_body,
    out_type=jax.ShapeDtypeStruct((_NC, _NPAD), jnp.float32),
    mesh=_mesh,
    scratch_types=[
        pltpu.VMEM_SHARED((_NPAD,), jnp.float32),
        pltpu.VMEM((_NCH, _CB), jnp.int32),
        pltpu.VMEM((_CB,), jnp.float32),
        pltpu.VMEM((_RPT,), jnp.float32),
    ],
    compiler_params=pltpu.CompilerParams(use_tc_tiling_on_sc=False),
)


def _edge_body(z_hbm, src_hbm, dst_hbm, out_hbm,
               ssh, src_v, dst_v, g0, g1, g2, g3, g4, g5, zb,
               semg, sems):
    c = lax.axis_index("c")
    s = lax.axis_index("s")
    w = c * _NS + s
    zeros16 = jnp.zeros((16,), jnp.float32)
    for r in range(_ZR):
        for q in range(_C // 16):
            zb[r, pl.ds(q * 16, 16)] = zeros16
    for blk in range(_RPT // _ZR):
        pltpu.sync_copy(zb, ssh.at[pl.ds(s * _RPT + blk * _ZR, _ZR)])
    pltpu.sync_copy(src_hbm.at[w], src_v)
    pltpu.sync_copy(dst_hbm.at[w], dst_v)
    plsc.subcore_barrier()
    gbufs = (g0, g1, g2, g3, g4, g5)
    nb = len(gbufs)
    gq = 3

    def gfire(j):
        return pltpu.async_copy(z_hbm.at[src_v.at[j]], gbufs[j % nb], semg)

    def sfire(j):
        return pltpu.async_copy(gbufs[j % nb], ssh.at[dst_v.at[j]], sems,
                                add=True)

    gd = {j: gfire(j) for j in range(min(gq, _NCH))}
    sd = {}
    for j in range(_NCH):
        gd.pop(j).wait()
        if j + gq < _NCH:
            if j + gq - nb >= 0:
                sd.pop(j + gq - nb).wait()
            gd[j + gq] = gfire(j + gq)
        sd[j] = sfire(j)
    for j in sorted(sd):
        sd.pop(j).wait()
    plsc.subcore_barrier()
    pltpu.sync_copy(ssh.at[pl.ds(s * _RPT, _RPT)],
                    out_hbm.at[c, pl.ds(s * _RPT, _RPT)])


_edge_call = pl.kernel(
    _edge_body,
    out_type=jax.ShapeDtypeStruct((_NC, _NPAD, _C), jnp.float32),
    mesh=_mesh,
    scratch_types=[
        pltpu.VMEM_SHARED((_NPAD, _C), jnp.float32),
        pltpu.VMEM((_NCH, _CB), jnp.int32),
        pltpu.VMEM((_NCH, _CB), jnp.int32),
        pltpu.VMEM((_CB, _C), jnp.float32),
        pltpu.VMEM((_CB, _C), jnp.float32),
        pltpu.VMEM((_CB, _C), jnp.float32),
        pltpu.VMEM((_CB, _C), jnp.float32),
        pltpu.VMEM((_CB, _C), jnp.float32),
        pltpu.VMEM((_CB, _C), jnp.float32),
        pltpu.VMEM((_ZR, _C), jnp.float32),
        pltpu.SemaphoreType.DMA,
        pltpu.SemaphoreType.DMA,
    ],
    compiler_params=pltpu.CompilerParams(use_tc_tiling_on_sc=False),
)


_RPW = _NPAD // _NW
_NBK = _RPW // _ZR


def _mix_body(p_hbm, d2_hbm, dv_hbm, acc_hbm, z_out, acc_out,
              pb0, pb1, d2b, dvb, acb, zwb, awb, semp, semz, *, ck):
    c = lax.axis_index("c")
    s = lax.axis_index("s")
    w = c * _NS + s

    def pfire(blk, slot):
        r0 = w * _RPW + blk * _ZR
        return (
            pltpu.async_copy(p_hbm.at[0, pl.ds(r0, _ZR)], pb0.at[slot], semp),
            pltpu.async_copy(p_hbm.at[1, pl.ds(r0, _ZR)], pb1.at[slot], semp),
            pltpu.async_copy(d2_hbm.at[pl.ds(r0, _ZR)], d2b.at[slot], semp),
            pltpu.async_copy(dv_hbm.at[pl.ds(r0, _ZR)], dvb.at[slot], semp),
            pltpu.async_copy(acc_hbm.at[pl.ds(r0, _ZR)], acb.at[slot], semp),
        )

    pend = pfire(0, 0)
    zpend = []
    for blk in range(_NBK):
        for d in pend:
            d.wait()
        slot = blk % 2
        if blk + 1 < _NBK:
            pend = pfire(blk + 1, 1 - slot)
        if blk >= 2:
            for d in zpend[blk - 2]:
                d.wait()

        def rowbody(r, carry, slot=slot):
            for q in range(_C // 16):
                sl = pl.ds(q * 16, 16)
                t = pb0[slot, r, sl] + pb1[slot, r, sl]
                zwb[slot, r, sl] = d2b[slot, r, sl] * t
                awb[slot, r, sl] = acb[slot, r, sl] + (
                    dvb[slot, r, sl] * t) * ck
            return carry

        lax.fori_loop(0, _ZR, rowbody, 0)
        r0 = w * _RPW + blk * _ZR
        zpend.append((
            pltpu.async_copy(zwb.at[slot], z_out.at[pl.ds(r0, _ZR)], semz),
            pltpu.async_copy(awb.at[slot], acc_out.at[pl.ds(r0, _ZR)], semz),
        ))
    for pair in zpend[-2:]:
        for d in pair:
            d.wait()


def _make_mix(ck):
    return pl.kernel(
        functools.partial(_mix_body, ck=ck),
        out_type=(
            jax.ShapeDtypeStruct((_NPAD, _C), jnp.float32),
            jax.ShapeDtypeStruct((_NPAD, _C), jnp.float32),
        ),
        mesh=_mesh,
        scratch_types=[
            pltpu.VMEM((2, _ZR, _C), jnp.float32),
            pltpu.VMEM((2, _ZR, _C), jnp.float32),
            pltpu.VMEM((2, _ZR, _C), jnp.float32),
            pltpu.VMEM((2, _ZR, _C), jnp.float32),
            pltpu.VMEM((2, _ZR, _C), jnp.float32),
            pltpu.VMEM((2, _ZR, _C), jnp.float32),
            pltpu.VMEM((2, _ZR, _C), jnp.float32),
            pltpu.SemaphoreType.DMA,
            pltpu.SemaphoreType.DMA,
        ],
        compiler_params=pltpu.CompilerParams(use_tc_tiling_on_sc=False),
    )


_mix_calls = [_make_mix(ck) for ck in _CK]


def _prep_body(feat_ref, wt_ref, degp_ref, b_ref,
               z0_ref, d2ex_ref, dvex_ref, acc0_ref):
    deg = jnp.maximum(degp_ref[0] + degp_ref[1], 1.0)
    dinv = lax.rsqrt(deg)
    y0 = jnp.dot(feat_ref[...], wt_ref[...], preferred_element_type=jnp.float32)
    z0_ref[...] = y0 * dinv
    zc = jnp.zeros((_NPAD, _C), jnp.float32)
    d2ex_ref[...] = zc + 1.0 / deg
    dvex_ref[...] = zc + dinv
    acc0_ref[...] = _CF * y0 + b_ref[...]


def _prep_call(feat_pad, wt, degp3, b2d):
    return pl.pallas_call(
        _prep_body,
        out_shape=(
            jax.ShapeDtypeStruct((_NPAD, _C), jnp.float32),
            jax.ShapeDtypeStruct((_NPAD, _C), jnp.float32),
            jax.ShapeDtypeStruct((_NPAD, _C), jnp.float32),
            jax.ShapeDtypeStruct((_NPAD, _C), jnp.float32),
        ),
    )(feat_pad, wt, degp3, b2d)


@jax.jit
def kernel(feat, edge_index, W, b):
    feat_pad = jnp.zeros((_NPAD, _D), jnp.float32).at[:_N].set(feat)
    loop = jnp.arange(_N, dtype=jnp.int32)
    npadrows = _NPAD - _N
    e_in = edge_index.shape[1]
    padi = _N + (jnp.arange(_EPAD - _N - e_in, dtype=jnp.int32) % npadrows)
    src = jnp.concatenate([edge_index[0], loop, padi]).reshape(_NW, _NCH, _CB)
    dst = jnp.concatenate([edge_index[1], loop, padi]).reshape(_NW, _NCH, _CB)

    degp = _deg_call(dst)
    z0, d2ex, dvex, acc = _prep_call(feat_pad, W.T, degp[:, :, None],
                                     jnp.reshape(b, (1, _C)))

    z = z0
    for k in range(_K):
        s_k = _edge_call(z, src, dst)
        z, acc = _mix_calls[k](s_k, d2ex, dvex, acc)

    return acc[:_N]
